# baseline (speedup 1.0000x reference)
import jax
import jax.numpy as jnp
from jax.experimental import pallas as pl
from jax.experimental.pallas import tpu as pltpu


def _halfblock(x, tgt, lens, t0):
    Th, Bb, Vb = x.shape
    m = jnp.max(x, axis=-1)
    idx = jax.lax.broadcasted_iota(jnp.int32, x.shape, 2).astype(jnp.float32)
    cand = jnp.where(x == m[..., None], idx, float(Vb))
    pred = jnp.min(cand, axis=-1)
    tids = t0 + jax.lax.broadcasted_iota(jnp.int32, (Th, Bb), 0)
    mask = tids < (lens - 1)
    corr = jnp.logical_and(pred == tgt.astype(jnp.float32), mask)
    c = jnp.sum(corr.astype(jnp.float32))
    v = jnp.sum(mask.astype(jnp.float32))
    return c, v


def _body(lens_ref, x1_ref, x2_ref, tok_ref, nxt_ref, out_ref, acc_ref):
    i = pl.program_id(0)

    @pl.when(i == 0)
    def _init():
        acc_ref[0] = 0.0
        acc_ref[1] = 0.0

    Th = x1_ref.shape[0]
    lens = lens_ref[...]
    tgt = jnp.concatenate([tok_ref[1:], nxt_ref[:1]], axis=0)
    c1, v1 = _halfblock(x1_ref[...], tgt[:Th], lens, i * 2 * Th)
    c2, v2 = _halfblock(x2_ref[...], tgt[Th:], lens, i * 2 * Th + Th)
    acc_ref[0] += c1 + c2
    acc_ref[1] += v1 + v2

    @pl.when(i == pl.num_programs(0) - 1)
    def _fini():
        out_ref[...] = jnp.full((1, 128), acc_ref[0] / acc_ref[1],
                                dtype=jnp.float32)


def kernel(outputs, tokens, tokens_lens):
    T, B, V = outputs.shape
    Tb = 32
    Th = Tb // 2
    n = T // Tb
    lens2d = tokens_lens.reshape(1, B)

    acc = pl.pallas_call(
        _body,
        grid=(n,),
        in_specs=[
            pl.BlockSpec((1, B), lambda i: (0, 0)),
            pl.BlockSpec((Th, B, V), lambda i: (2 * i, 0, 0)),
            pl.BlockSpec((Th, B, V), lambda i: (2 * i + 1, 0, 0)),
            pl.BlockSpec((Tb, B), lambda i: (i, 0)),
            pl.BlockSpec((Tb, B), lambda i: (jnp.minimum(i + 1, n - 1), 0)),
        ],
        out_specs=pl.BlockSpec((1, 128), lambda i: (0, 0)),
        out_shape=jax.ShapeDtypeStruct((1, 128), jnp.float32),
        scratch_shapes=[pltpu.SMEM((2,), jnp.float32)],
        compiler_params=pltpu.CompilerParams(
            dimension_semantics=("arbitrary",),
        ),
    )(lens2d, outputs, outputs, tokens, tokens)
    return acc[0, 0]


# speedup vs baseline: 1.0553x; 1.0553x over previous
import jax
import jax.numpy as jnp
from jax.experimental import pallas as pl
from jax.experimental.pallas import tpu as pltpu

_NS = 4


def _subblock(x, tgt, lens, t0):
    Th, Bb, Vb = x.shape
    m = jnp.max(x, axis=-1)
    idx = jax.lax.broadcasted_iota(jnp.int32, x.shape, 2).astype(jnp.float32)
    cand = jnp.where(x == m[..., None], idx, float(Vb))
    pred = jnp.min(cand, axis=-1)
    tids = t0 + jax.lax.broadcasted_iota(jnp.int32, (Th, Bb), 0)
    mask = tids < (lens - 1)
    corr = jnp.logical_and(pred == tgt.astype(jnp.float32), mask)
    c = jnp.sum(corr.astype(jnp.float32))
    v = jnp.sum(mask.astype(jnp.float32))
    return c, v


def _body(lens_ref, *refs):
    x_refs = refs[:_NS]
    tok_ref, nxt_ref, out_ref, acc_ref = refs[_NS:]
    i = pl.program_id(0)

    @pl.when(i == 0)
    def _init():
        acc_ref[0] = 0.0
        acc_ref[1] = 0.0

    Th = x_refs[0].shape[0]
    lens = lens_ref[...]
    tgt = jnp.concatenate([tok_ref[1:], nxt_ref[:1]], axis=0)
    c = 0.0
    v = 0.0
    for s in range(_NS):
        cs, vs = _subblock(x_refs[s][...], tgt[s * Th:(s + 1) * Th], lens,
                           (i * _NS + s) * Th)
        c += cs
        v += vs
    acc_ref[0] += c
    acc_ref[1] += v

    @pl.when(i == pl.num_programs(0) - 1)
    def _fini():
        out_ref[...] = jnp.full((1, 128), acc_ref[0] / acc_ref[1],
                                dtype=jnp.float32)


def kernel(outputs, tokens, tokens_lens):
    T, B, V = outputs.shape
    Tb = 64
    Th = Tb // _NS
    n = T // Tb
    lens2d = tokens_lens.reshape(1, B)

    def x_spec(s):
        return pl.BlockSpec((Th, B, V), lambda i, s=s: (_NS * i + s, 0, 0))

    acc = pl.pallas_call(
        _body,
        grid=(n,),
        in_specs=[pl.BlockSpec((1, B), lambda i: (0, 0))]
        + [x_spec(s) for s in range(_NS)]
        + [
            pl.BlockSpec((Tb, B), lambda i: (i, 0)),
            pl.BlockSpec((Tb, B), lambda i: (jnp.minimum(i + 1, n - 1), 0)),
        ],
        out_specs=pl.BlockSpec((1, 128), lambda i: (0, 0)),
        out_shape=jax.ShapeDtypeStruct((1, 128), jnp.float32),
        scratch_shapes=[pltpu.SMEM((2,), jnp.float32)],
        compiler_params=pltpu.CompilerParams(
            dimension_semantics=("arbitrary",),
        ),
    )(lens2d, *([outputs] * _NS), tokens, tokens)
    return acc[0, 0]
